# baseline (speedup 1.0000x reference)
import functools

import jax
import jax.numpy as jnp
from jax import lax
from jax.experimental import pallas as pl
from jax.experimental.pallas import tpu as pltpu
from jax.experimental.pallas import tpu_sc as plsc

N = 10000
NP = 10000
E = 320000
NC = 2
NS = 16
EPT = E // (NC * NS)
ROWS_PER_TILE = NP // NS
C = 125
CH = EPT // C
NB = 8


def _make_sc_seg_sum(F, gather):
    mesh = plsc.VectorSubcoreMesh(core_axis_name="c", subcore_axis_name="s")
    nrows = NB if gather else 1
    dt = jnp.bfloat16 if gather else jnp.float32
    VW = 32 if gather else 16

    scratch = (
        [pltpu.VMEM_SHARED((NP, F), dt)]
        + [pltpu.VMEM((CH, C), jnp.int32)]
        + ([pltpu.VMEM((CH, C), jnp.int32)] if gather else [])
        + [pltpu.VMEM((C, F), dt) for _ in range(nrows)]
        + [pltpu.SemaphoreType.DMA for _ in range(NB * (2 if gather else 1))]
    )

    def body(*refs):
        it = iter(refs)
        if gather:
            y_hbm = next(it)
        edge2 = next(it)
        out_hbm = next(it)
        acc = next(it)
        didx = next(it)
        sidx = next(it) if gather else None
        rows = [next(it) for _ in range(nrows)]
        ssem = [next(it) for _ in range(NB)]
        gsem = [next(it) for _ in range(NB)] if gather else None

        cid = lax.axis_index("c")
        sid = lax.axis_index("s")

        zero = jnp.zeros((VW,), dt)
        one = jnp.ones((VW,), dt)

        def fill(buf, val):
            def fill_row(r, carry):
                for j in range(F // VW):
                    buf[r, pl.ds(j * VW, VW)] = val
                return carry

            lax.fori_loop(0, C, fill_row, 0)

        fill(rows[0], zero)
        rbase = sid * ROWS_PER_TILE
        for t in range(ROWS_PER_TILE // C):
            pltpu.sync_copy(rows[0], acc.at[pl.ds(rbase + t * C, C)])
        remz = ROWS_PER_TILE % C
        if remz:
            pltpu.sync_copy(
                rows[0].at[pl.ds(0, remz)],
                acc.at[pl.ds(rbase + (ROWS_PER_TILE // C) * C, remz)])
        if not gather:
            fill(rows[0], one)
        plsc.subcore_barrier()

        cbase = (cid * NS + sid) * CH
        pltpu.sync_copy(edge2.at[1, pl.ds(cbase, CH)], didx)
        if gather:
            pltpu.sync_copy(edge2.at[0, pl.ds(cbase, CH)], sidx)

        def sstart(i, r, rr):
            pltpu.async_copy(rows[rr], acc.at[didx.at[i]], ssem[r], add=True)

        def swait(r, rr):
            pltpu.make_async_copy(rows[rr], acc.at[didx.at[0]],
                                  ssem[r]).wait()

        if gather:
            def gstart(i, r):
                pltpu.async_copy(y_hbm.at[sidx.at[i]], rows[r], gsem[r])

            def gwait(r):
                pltpu.make_async_copy(y_hbm.at[sidx.at[0]], rows[r],
                                      gsem[r]).wait()

            GLAG = NB - 2

            def step(i, r):
                gwait(r)
                sstart(i, r, r)
                rp = (r + GLAG) % NB
                swait(rp, rp)
                gstart(i + GLAG, rp)

            for i in range(GLAG):
                gstart(i, i)
            for i in range(NB - GLAG):
                gwait(i)
                sstart(i, i, i)
                gstart(i + GLAG, i + GLAG)
            first_u = NB - GLAG
            nblocks = (CH - NB) // NB

            def kblock(k, carry):
                b = first_u + NB * k
                for j in range(NB):
                    step(b + j, (first_u + j) % NB)
                return carry

            lax.fori_loop(0, nblocks, kblock, 0)
            for i in range(first_u + NB * nblocks, CH - GLAG):
                step(i, i % NB)
            for i in range(CH - GLAG, CH):
                gwait(i % NB)
                sstart(i, i % NB, i % NB)
            for i in range(CH - NB, CH):
                swait(i % NB, i % NB)
        else:
            sstart(0, 0, 0)
            sstart(1, 1, 0)
            npairs = (CH - 2) // 2

            def kblock(k, carry):
                i = 2 + 2 * k
                swait(0, 0)
                sstart(i, 0, 0)
                swait(1, 0)
                sstart(i + 1, 1, 0)
                return carry

            lax.fori_loop(0, npairs, kblock, 0)
            for i in range(2 + 2 * npairs, CH):
                p = i & 1
                swait(p, 0)
                sstart(i, p, 0)
            swait(1, 0)
            swait(0, 0)

        plsc.subcore_barrier()

        pltpu.sync_copy(acc.at[pl.ds(rbase, ROWS_PER_TILE)],
                        out_hbm.at[cid, pl.ds(rbase, ROWS_PER_TILE)])

    out_type = jax.ShapeDtypeStruct((NC, NP, F), dt)
    return pl.kernel(body, mesh=mesh, out_type=out_type,
                     scratch_types=scratch,
                     compiler_params=pltpu.CompilerParams(
                         use_tc_tiling_on_sc=False))


_sc_seg_sum = functools.cache(_make_sc_seg_sum)

_R = 2000


def _dinv(dgp):
    deg = dgp[0, :, :1] + dgp[1, :, :1]
    return jnp.where(deg > 0, lax.rsqrt(deg), 0.0)


def _tc_first_body(x_ref, w_ref, dgp_ref, y_ref):
    d = _dinv(dgp_ref[...])
    y = d * jnp.dot(x_ref[...], w_ref[...],
                    preferred_element_type=jnp.float32)
    y_ref[...] = y.astype(jnp.bfloat16)


def _psum(s_ref):
    return (s_ref[0].astype(jnp.float32) + s_ref[1].astype(jnp.float32))


def _tc_mid_body(s_ref, dgp_ref, b_ref, w_ref, y_ref):
    d = _dinv(dgp_ref[...])
    h = jax.nn.relu(d * _psum(s_ref) + b_ref[...])
    y = d * jnp.dot(h, w_ref[...], preferred_element_type=jnp.float32)
    y_ref[...] = y.astype(jnp.bfloat16)


def _tc_last_body(s_ref, dgp_ref, b_ref, y_ref):
    d = _dinv(dgp_ref[...])
    y_ref[...] = jax.nn.relu(d * _psum(s_ref) + b_ref[...])


def _row_spec(F):
    return pl.BlockSpec((_R, F), lambda i: (i, 0))


def _pair_spec(F):
    return pl.BlockSpec((2, _R, F), lambda i: (0, i, 0))


def _whole_spec(shape):
    return pl.BlockSpec(shape, lambda i: tuple(0 for _ in shape))


def _tc_first(x, w, dgp):
    fin, fout = w.shape
    return pl.pallas_call(
        _tc_first_body,
        grid=(N // _R,),
        in_specs=[_row_spec(fin), _whole_spec((fin, fout)), _pair_spec(16)],
        out_specs=_row_spec(fout),
        out_shape=jax.ShapeDtypeStruct((N, fout), jnp.bfloat16),
    )(x, w, dgp)


def _tc_mid(s, dgp, b, w):
    fin, fout = w.shape
    return pl.pallas_call(
        _tc_mid_body,
        grid=(N // _R,),
        in_specs=[_pair_spec(fin), _pair_spec(16),
                  _whole_spec((1, fin)), _whole_spec((fin, fout))],
        out_specs=_row_spec(fout),
        out_shape=jax.ShapeDtypeStruct((N, fout), jnp.bfloat16),
    )(s, dgp, b, w)


def _tc_last(s, dgp, b):
    fout = s.shape[2]
    return pl.pallas_call(
        _tc_last_body,
        grid=(N // _R,),
        in_specs=[_pair_spec(fout), _pair_spec(16), _whole_spec((1, fout))],
        out_specs=_row_spec(fout),
        out_shape=jax.ShapeDtypeStruct((N, fout), jnp.float32),
    )(s, dgp, b)


@jax.jit
def kernel(features, edge_index, W0, b0, W1, b1, W2, b2):
    edge2 = edge_index.astype(jnp.int32).reshape(2, E // C, C)

    degp = _sc_seg_sum(16, False)(edge2)
    y0 = _tc_first(features, W0, degp)
    s0 = _sc_seg_sum(128, True)(y0, edge2)
    y1 = _tc_mid(s0, degp, b0.reshape(1, -1), W1)
    s1 = _sc_seg_sum(128, True)(y1, edge2)
    y2 = _tc_mid(s1, degp, b1.reshape(1, -1), W2)
    s2 = _sc_seg_sum(64, True)(y2, edge2)
    return _tc_last(s2, degp, b2.reshape(1, -1))
